# baseline (speedup 1.0000x reference)
import functools

import jax
import jax.numpy as jnp
from jax import lax
from jax.experimental import pallas as pl
from jax.experimental.pallas import tpu as pltpu
from jax.experimental.pallas import tpu_sc as plsc

N = 10000
N_PAD = 10240
P = 2000
P_PAD = 2400
D = 128
E = 320000
TOPO_E_MEAN = 2.5
TOPO_E_STD = 1.2
ETA_MEAN = 0.0
ETA_STD = 1.5

NODE_BLK = 1024
P_BLK = 400
N_BLK = 2048


def _node_kernel(nh, sc, en, tr, wk1, wk2, wke, wkt, bk, wv1, wv2, wve, wvt, bv,
                 k_out, v_out):
    logE = jnp.log(en[...])
    f32 = jnp.float32
    dot = lambda a, b: jax.lax.dot_general(a, b, (((1,), (0,)), ((), ())),
                                           preferred_element_type=f32)
    kv = dot(nh[...], wk1[...]) + dot(sc[...], wk2[...])
    kv += logE * wke[...] + tr[...] * wkt[...] + bk[...]
    k_out[...] = kv
    vv = dot(nh[...], wv1[...]) + dot(sc[...], wv2[...])
    vv += logE * wve[...] + tr[...] * wvt[...] + bv[...]
    v_out[...] = vv


def _node_projections(nh, sc_pad, en, tr, wk1, wk2, wke, wkt, bk,
                      wv1, wv2, wve, wvt, bv):
    grid = (N_PAD // NODE_BLK,)
    row = lambda i: (i, 0)
    full = lambda i: (0, 0)
    return pl.pallas_call(
        _node_kernel,
        grid=grid,
        in_specs=[
            pl.BlockSpec((NODE_BLK, D), row),
            pl.BlockSpec((NODE_BLK, 32), row),
            pl.BlockSpec((NODE_BLK, 1), row),
            pl.BlockSpec((NODE_BLK, 1), row),
            pl.BlockSpec((D, D), full),
            pl.BlockSpec((32, D), full),
            pl.BlockSpec((1, D), full),
            pl.BlockSpec((1, D), full),
            pl.BlockSpec((1, D), full),
            pl.BlockSpec((D, D), full),
            pl.BlockSpec((32, D), full),
            pl.BlockSpec((1, D), full),
            pl.BlockSpec((1, D), full),
            pl.BlockSpec((1, D), full),
        ],
        out_specs=[
            pl.BlockSpec((NODE_BLK, D), row),
            pl.BlockSpec((NODE_BLK, D), row),
        ],
        out_shape=[
            jax.ShapeDtypeStruct((N_PAD, D), jnp.float32),
            jax.ShapeDtypeStruct((N_PAD, D), jnp.float32),
        ],
    )(nh, sc_pad, en, tr, wk1, wk2, wke, wkt, bk, wv1, wv2, wve, wvt, bv)


def _fused_kernel(cmat, kmat, vmat, ph, pg, wq1, wq2, bq,
                  w_iht, w_hht, b_ih, b_hh, ln_g, ln_b,
                  w_m1, b_m1, w_m2, b_m2, rms_w, lin_w,
                  out, q_scr, acc_scr):
    nb = pl.program_id(1)
    f32 = jnp.float32
    dot = lambda a, b: jax.lax.dot_general(a, b, (((1,), (0,)), ((), ())),
                                           preferred_element_type=f32)

    @pl.when(nb == 0)
    def _init():
        q_scr[...] = dot(ph[...], wq1[...]) + dot(pg[...], wq2[...]) + bq[...]
        acc_scr[...] = jnp.zeros((P_BLK, D), f32)

    ksl = pl.ds(nb * N_BLK, N_BLK)
    s = jax.lax.dot_general(q_scr[...], kmat[ksl, :], (((1,), (1,)), ((), ())),
                            preferred_element_type=f32)
    s = jnp.maximum(s * 0.1, 0.0) * cmat[...].astype(f32)
    acc_scr[...] += dot(s, vmat[ksl, :])

    @pl.when(nb == pl.num_programs(1) - 1)
    def _epilogue():
        w = acc_scr[...]
        rms = w * jax.lax.rsqrt(jnp.mean(w * w, axis=-1, keepdims=True) + 1e-6)
        rms = rms * rms_w[...]
        ws = rms * jax.nn.sigmoid(w * lin_w[...])
        gi = dot(ws, w_iht[...]) + b_ih[...]
        gh = dot(ph[...], w_hht[...]) + b_hh[...]
        r = jax.nn.sigmoid(gi[:, 0:D] + gh[:, 0:D])
        z = jax.nn.sigmoid(gi[:, D:2 * D] + gh[:, D:2 * D])
        nn = jnp.tanh(gi[:, 2 * D:3 * D] + r * gh[:, 2 * D:3 * D])
        h = (1.0 - z) * nn + z * ph[...]
        mu = jnp.mean(h, axis=-1, keepdims=True)
        var = jnp.mean((h - mu) ** 2, axis=-1, keepdims=True)
        ln = (h - mu) * jax.lax.rsqrt(var + 1e-5) * ln_g[...] + ln_b[...]
        mlp = dot(jnp.maximum(dot(ln, w_m1[...]) + b_m1[...], 0.0), w_m2[...])
        out[...] = ph[...] + mlp + b_m2[...]


def _fused_stage(cmat, kmat, vmat, ph, pg, wq1, wq2, bq,
                 w_iht, w_hht, b_ih, b_hh, ln_g, ln_b,
                 w_m1, b_m1, w_m2, b_m2, rms_w, lin_w):
    grid = (P // P_BLK, N_PAD // N_BLK)
    pblk = lambda p, n: (p, 0)
    nrow = lambda p, n: (n, 0)
    full = lambda p, n: (0, 0)
    return pl.pallas_call(
        _fused_kernel,
        grid=grid,
        in_specs=[
            pl.BlockSpec((P_BLK, N_BLK), lambda p, n: (p, n)),
            pl.BlockSpec((N_PAD, D), full),
            pl.BlockSpec((N_PAD, D), full),
            pl.BlockSpec((P_BLK, D), pblk),
            pl.BlockSpec((P_BLK, D), pblk),
            pl.BlockSpec((D, D), full),
            pl.BlockSpec((D, D), full),
            pl.BlockSpec((1, D), full),
            pl.BlockSpec((D, 3 * D), full),
            pl.BlockSpec((D, 3 * D), full),
            pl.BlockSpec((1, 3 * D), full),
            pl.BlockSpec((1, 3 * D), full),
            pl.BlockSpec((1, D), full),
            pl.BlockSpec((1, D), full),
            pl.BlockSpec((D, 64), full),
            pl.BlockSpec((1, 64), full),
            pl.BlockSpec((64, D), full),
            pl.BlockSpec((1, D), full),
            pl.BlockSpec((1, D), full),
            pl.BlockSpec((1, D), full),
        ],
        out_specs=pl.BlockSpec((P_BLK, D), pblk),
        out_shape=jax.ShapeDtypeStruct((P, D), jnp.float32),
        scratch_shapes=[
            pltpu.VMEM((P_BLK, D), jnp.float32),
            pltpu.VMEM((P_BLK, D), jnp.float32),
        ],
    )(cmat, kmat, vmat, ph, pg, wq1, wq2, bq, w_iht, w_hht, b_ih, b_hh,
      ln_g, ln_b, w_m1, b_m1, w_m2, b_m2, rms_w, lin_w)


G_ROWS = 8
N_GROUP = 272
GPT = 8
E_WIN = 1024
A_SLICE = E // 16

_sc_mesh = plsc.VectorSubcoreMesh(core_axis_name="c", subcore_axis_name="s")


@functools.partial(
    pl.kernel,
    out_type=jax.ShapeDtypeStruct((P_PAD, N_PAD), jnp.float32),
    mesh=_sc_mesh,
    compiler_params=pltpu.CompilerParams(needs_layout_passes=False),
    scratch_types=[
        pltpu.VMEM((2, 128), jnp.float32),
        pltpu.VMEM((16, 2, 128), jnp.float32),
        pltpu.VMEM((N_GROUP,), jnp.int32),
        pltpu.VMEM((E_WIN,), jnp.int32),
        pltpu.VMEM((E_WIN,), jnp.int32),
        pltpu.VMEM((G_ROWS, N_PAD), jnp.float32),
        pltpu.VMEM_SHARED((16, 2, 128), jnp.float32),
        pltpu.VMEM_SHARED((N_GROUP,), jnp.int32),
    ],
)
def _c_build_sc(src_hbm, dst_hbm, c_hbm, hist_l, hist_all, rs_l,
                srcb, dstb, cbuf, hist_sh, rs_sh):
    f32 = jnp.float32
    cid = lax.axis_index("c")
    sid = lax.axis_index("s")
    wid = cid * 16 + sid
    ones = jnp.full((16,), 1.0, f32)
    zeros16 = jnp.zeros((16,), f32)

    for hr in range(2):
        for v in range(8):
            hist_l[hr, pl.ds(v * 16, 16)] = zeros16
    a0 = sid * A_SLICE
    n_full, rem = divmod(A_SLICE, E_WIN)
    for w in range(n_full + (1 if rem else 0)):
        wlen = E_WIN if w < n_full else rem
        a_start = pl.multiple_of(a0 + w * E_WIN, 8)
        pltpu.sync_copy(dst_hbm.at[pl.ds(a_start, wlen)], dstb.at[pl.ds(0, wlen)])

        def abody(i, _):
            d = dstb[pl.ds(i * 16, 16)]
            g = lax.shift_right_logical(d, 3)
            gr = lax.shift_right_logical(g, 7)
            gc = jnp.bitwise_and(g, 127)
            plsc.addupdate_scatter(hist_l, [gr, gc], ones)
            return 0

        lax.fori_loop(0, wlen // 16, abody, 0)
    pltpu.sync_copy(hist_l, hist_sh.at[sid])
    plsc.subcore_barrier()

    @pl.when(sid == 0)
    def _reduce_scan():
        pltpu.sync_copy(hist_sh, hist_all)
        carry = jnp.int32(0)
        for v in range(16):
            hr, hc = v >> 3, (v & 7) * 16
            tot = hist_all[0, hr, pl.ds(hc, 16)]
            for r in range(1, 16):
                tot = tot + hist_all[r, hr, pl.ds(hc, 16)]
            incl = plsc.cumsum(tot)
            excl = (incl - tot).astype(jnp.int32) + carry
            rs_l[pl.ds(v * 16, 16)] = excl
            carry = carry + jnp.sum(tot).astype(jnp.int32)
        rs_l[pl.ds(256, 16)] = jnp.full((16,), 1, jnp.int32) * carry
        pltpu.sync_copy(rs_l, rs_sh)

    plsc.subcore_barrier()
    pltpu.sync_copy(rs_sh, rs_l)

    for j in range(GPT):
        k = wid * GPT + j
        lo = k * G_ROWS

        @pl.when(lo < P)
        def _group():
            s0 = rs_l[pl.ds(k, 16)][0]
            e0 = rs_l[pl.ds(k + 1, 16)][0]
            for r in range(G_ROWS):
                def zbody(cb, _):
                    for u in range(16):
                        cbuf[r, pl.ds((cb * 16 + u) * 16, 16)] = zeros16
                    return 0
                lax.fori_loop(0, N_PAD // 256, zbody, 0)
            base = jnp.bitwise_and(s0, jnp.int32(-8))
            n_win = lax.shift_right_logical(e0 - base + (E_WIN - 1), 10)

            def wbody(w, _):
                start = pl.multiple_of(base + w * E_WIN, 8)
                pltpu.sync_copy(src_hbm.at[pl.ds(start, E_WIN)], srcb)
                pltpu.sync_copy(dst_hbm.at[pl.ds(start, E_WIN)], dstb)

                def ebody(i, _):
                    sv = srcb[pl.ds(i * 16, 16)]
                    dv = dstb[pl.ds(i * 16, 16)]
                    rr = dv - lo
                    m = jnp.logical_and(dv >= lo, dv < lo + G_ROWS)
                    plsc.addupdate_scatter(cbuf, [rr, sv], ones, mask=m)
                    return 0

                lax.fori_loop(0, E_WIN // 16, ebody, 0)
                return 0

            lax.fori_loop(0, n_win, wbody, 0)
            pltpu.sync_copy(cbuf, c_hbm.at[pl.ds(lo, G_ROWS), :])


def _edge_counts(edge_src, edge_dst):
    src_p = jnp.pad(edge_src, (0, E_WIN))
    dst_p = jnp.pad(edge_dst, (0, E_WIN), constant_values=1 << 20)
    return _c_build_sc(src_p, dst_p)


def kernel(node_hidden, energy, node_scalars, isTrack, particle_hidden,
           particle_global, edge_dR, edge_src, edge_dst, W_key, b_key,
           W_val, b_val, W_q, b_q, W_ih, W_hh, b_ih, b_hh, ln_g, ln_b,
           W_m1, b_m1, W_m2, b_m2, rms_w, lin_w):
    f32 = jnp.float32

    scale = jnp.ones((27,), f32)
    shift = jnp.zeros((27,), f32)
    scale = scale.at[0].set(1.0 / TOPO_E_STD).at[1].set(1.0 / ETA_STD)
    shift = shift.at[0].set(TOPO_E_MEAN).at[1].set(ETA_MEAN)
    scale = jax.lax.dynamic_update_slice(
        scale, jnp.full((6,), 1.0 / ETA_STD, f32), (4,))
    shift = jax.lax.dynamic_update_slice(
        shift, jnp.full((6,), ETA_MEAN, f32), (4,))

    def fold(W, b):
        Wskip = W[D:] * scale[:, None]
        b2 = b - (shift * scale) @ W[D:]
        w_e = Wskip[0:1]
        w_t = Wskip[22:23]
        Wsc = jnp.concatenate([Wskip[1:22], Wskip[23:27],
                               jnp.zeros((7, W.shape[1]), f32)], axis=0)
        return W[:D], Wsc, w_e, w_t, b2[None, :]

    def pad_out(x, width=D):
        return jnp.pad(x, ((0, 0), (0, width - x.shape[1])))

    wk1, wk2, wke, wkt, bk = (pad_out(a) for a in fold(W_key, b_key))
    wv1, wv2, wve, wvt, bv = fold(W_val, b_val)

    pad_n = ((0, N_PAD - N), (0, 0))
    nh_p = jnp.pad(node_hidden, pad_n)
    sc_p = jnp.pad(node_scalars, ((0, N_PAD - N), (0, 32 - 25)))
    en_p = jnp.pad(energy[:, None], pad_n, constant_values=1.0)
    tr_p = jnp.pad(isTrack[:, None], pad_n)

    kmat, vmat = _node_projections(nh_p, sc_p, en_p, tr_p,
                                   wk1, wk2, wke, wkt, bk,
                                   wv1, wv2, wve, wvt, bv)

    cmat = _edge_counts(edge_src, edge_dst)

    wq1 = pad_out(W_q[:D])
    wq2 = pad_out(W_q[D:])
    bq = pad_out(b_q[None, :])

    out = _fused_stage(cmat, kmat, vmat, particle_hidden, particle_global,
                       wq1, wq2, bq, W_ih.T, W_hh.T, b_ih[None, :],
                       b_hh[None, :], ln_g[None, :], ln_b[None, :],
                       W_m1, b_m1[None, :], W_m2, b_m2[None, :],
                       rms_w[None, :], lin_w)
    return out


# speedup vs baseline: 1.0280x; 1.0280x over previous
import functools

import jax
import jax.numpy as jnp
from jax import lax
from jax.experimental import pallas as pl
from jax.experimental.pallas import tpu as pltpu
from jax.experimental.pallas import tpu_sc as plsc

N = 10000
N_PAD = 10240
P = 2000
P_PAD = 2400
D = 128
E = 320000
TOPO_E_MEAN = 2.5
TOPO_E_STD = 1.2
ETA_MEAN = 0.0
ETA_STD = 1.5

NODE_BLK = 1024
P_BLK = 400
N_BLK = 2048


def _node_kernel(nh, sc, en, tr, wk1, wk2, wke, wkt, bk, wv1, wv2, wve, wvt, bv,
                 k_out, v_out):
    logE = jnp.log(en[...])
    f32 = jnp.float32
    dot = lambda a, b: jax.lax.dot_general(a, b, (((1,), (0,)), ((), ())),
                                           preferred_element_type=f32)
    kv = dot(nh[...], wk1[...]) + dot(sc[...], wk2[...])
    kv += logE * wke[...] + tr[...] * wkt[...] + bk[...]
    k_out[...] = kv.astype(jnp.bfloat16)
    vv = dot(nh[...], wv1[...]) + dot(sc[...], wv2[...])
    vv += logE * wve[...] + tr[...] * wvt[...] + bv[...]
    v_out[...] = vv.astype(jnp.bfloat16)


def _node_projections(nh, sc_pad, en, tr, wk1, wk2, wke, wkt, bk,
                      wv1, wv2, wve, wvt, bv):
    grid = (N_PAD // NODE_BLK,)
    row = lambda i: (i, 0)
    full = lambda i: (0, 0)
    return pl.pallas_call(
        _node_kernel,
        grid=grid,
        in_specs=[
            pl.BlockSpec((NODE_BLK, D), row),
            pl.BlockSpec((NODE_BLK, 32), row),
            pl.BlockSpec((NODE_BLK, 1), row),
            pl.BlockSpec((NODE_BLK, 1), row),
            pl.BlockSpec((D, D), full),
            pl.BlockSpec((32, D), full),
            pl.BlockSpec((1, D), full),
            pl.BlockSpec((1, D), full),
            pl.BlockSpec((1, D), full),
            pl.BlockSpec((D, D), full),
            pl.BlockSpec((32, D), full),
            pl.BlockSpec((1, D), full),
            pl.BlockSpec((1, D), full),
            pl.BlockSpec((1, D), full),
        ],
        out_specs=[
            pl.BlockSpec((NODE_BLK, D), row),
            pl.BlockSpec((NODE_BLK, D), row),
        ],
        out_shape=[
            jax.ShapeDtypeStruct((N_PAD, D), jnp.bfloat16),
            jax.ShapeDtypeStruct((N_PAD, D), jnp.bfloat16),
        ],
    )(nh, sc_pad, en, tr, wk1, wk2, wke, wkt, bk, wv1, wv2, wve, wvt, bv)


def _fused_kernel(cmat, kmat, vmat, ph, pg, wq1, wq2, bq,
                  w_iht, w_hht, b_ih, b_hh, ln_g, ln_b,
                  w_m1, b_m1, w_m2, b_m2, rms_w, lin_w,
                  out, q_scr, acc_scr):
    nb = pl.program_id(1)
    f32 = jnp.float32
    dot = lambda a, b: jax.lax.dot_general(a, b, (((1,), (0,)), ((), ())),
                                           preferred_element_type=f32)

    @pl.when(nb == 0)
    def _init():
        q_scr[...] = dot(ph[...], wq1[...]) + dot(pg[...], wq2[...]) + bq[...]
        acc_scr[...] = jnp.zeros((P_BLK, D), f32)

    ksl = pl.ds(nb * N_BLK, N_BLK)
    q16 = q_scr[...].astype(jnp.bfloat16)
    s = jax.lax.dot_general(q16, kmat[ksl, :], (((1,), (1,)), ((), ())),
                            preferred_element_type=f32)
    s = jnp.maximum(s * 0.1, 0.0) * cmat[...].astype(f32)
    acc_scr[...] += jax.lax.dot_general(
        s.astype(jnp.bfloat16), vmat[ksl, :], (((1,), (0,)), ((), ())),
        preferred_element_type=f32)

    @pl.when(nb == pl.num_programs(1) - 1)
    def _epilogue():
        w = acc_scr[...]
        rms = w * jax.lax.rsqrt(jnp.mean(w * w, axis=-1, keepdims=True) + 1e-6)
        rms = rms * rms_w[...]
        ws = rms * jax.nn.sigmoid(w * lin_w[...])
        gi = dot(ws, w_iht[...]) + b_ih[...]
        gh = dot(ph[...], w_hht[...]) + b_hh[...]
        r = jax.nn.sigmoid(gi[:, 0:D] + gh[:, 0:D])
        z = jax.nn.sigmoid(gi[:, D:2 * D] + gh[:, D:2 * D])
        nn = jnp.tanh(gi[:, 2 * D:3 * D] + r * gh[:, 2 * D:3 * D])
        h = (1.0 - z) * nn + z * ph[...]
        mu = jnp.mean(h, axis=-1, keepdims=True)
        var = jnp.mean((h - mu) ** 2, axis=-1, keepdims=True)
        ln = (h - mu) * jax.lax.rsqrt(var + 1e-5) * ln_g[...] + ln_b[...]
        mlp = dot(jnp.maximum(dot(ln, w_m1[...]) + b_m1[...], 0.0), w_m2[...])
        out[...] = ph[...] + mlp + b_m2[...]


def _fused_stage(cmat, kmat, vmat, ph, pg, wq1, wq2, bq,
                 w_iht, w_hht, b_ih, b_hh, ln_g, ln_b,
                 w_m1, b_m1, w_m2, b_m2, rms_w, lin_w):
    grid = (P // P_BLK, N_PAD // N_BLK)
    pblk = lambda p, n: (p, 0)
    nrow = lambda p, n: (n, 0)
    full = lambda p, n: (0, 0)
    return pl.pallas_call(
        _fused_kernel,
        grid=grid,
        in_specs=[
            pl.BlockSpec((P_BLK, N_BLK), lambda p, n: (p, n)),
            pl.BlockSpec((N_PAD, D), full),
            pl.BlockSpec((N_PAD, D), full),
            pl.BlockSpec((P_BLK, D), pblk),
            pl.BlockSpec((P_BLK, D), pblk),
            pl.BlockSpec((D, D), full),
            pl.BlockSpec((D, D), full),
            pl.BlockSpec((1, D), full),
            pl.BlockSpec((D, 3 * D), full),
            pl.BlockSpec((D, 3 * D), full),
            pl.BlockSpec((1, 3 * D), full),
            pl.BlockSpec((1, 3 * D), full),
            pl.BlockSpec((1, D), full),
            pl.BlockSpec((1, D), full),
            pl.BlockSpec((D, 64), full),
            pl.BlockSpec((1, 64), full),
            pl.BlockSpec((64, D), full),
            pl.BlockSpec((1, D), full),
            pl.BlockSpec((1, D), full),
            pl.BlockSpec((1, D), full),
        ],
        out_specs=pl.BlockSpec((P_BLK, D), pblk),
        out_shape=jax.ShapeDtypeStruct((P, D), jnp.float32),
        scratch_shapes=[
            pltpu.VMEM((P_BLK, D), jnp.float32),
            pltpu.VMEM((P_BLK, D), jnp.float32),
        ],
    )(cmat, kmat, vmat, ph, pg, wq1, wq2, bq, w_iht, w_hht, b_ih, b_hh,
      ln_g, ln_b, w_m1, b_m1, w_m2, b_m2, rms_w, lin_w)


G_ROWS = 8
N_GROUP = 272
GPT = 8
E_WIN = 1024
A_SLICE = E // 16

_sc_mesh = plsc.VectorSubcoreMesh(core_axis_name="c", subcore_axis_name="s")


@functools.partial(
    pl.kernel,
    out_type=jax.ShapeDtypeStruct((P_PAD, N_PAD), jnp.float32),
    mesh=_sc_mesh,
    compiler_params=pltpu.CompilerParams(needs_layout_passes=False),
    scratch_types=[
        pltpu.VMEM((2, 128), jnp.float32),
        pltpu.VMEM((16, 2, 128), jnp.float32),
        pltpu.VMEM((N_GROUP,), jnp.int32),
        pltpu.VMEM((E_WIN,), jnp.int32),
        pltpu.VMEM((E_WIN,), jnp.int32),
        pltpu.VMEM((G_ROWS, N_PAD), jnp.float32),
        pltpu.VMEM_SHARED((16, 2, 128), jnp.float32),
        pltpu.VMEM_SHARED((N_GROUP,), jnp.int32),
    ],
)
def _c_build_sc(src_hbm, dst_hbm, c_hbm, hist_l, hist_all, rs_l,
                srcb, dstb, cbuf, hist_sh, rs_sh):
    f32 = jnp.float32
    cid = lax.axis_index("c")
    sid = lax.axis_index("s")
    wid = cid * 16 + sid
    ones = jnp.full((16,), 1.0, f32)
    zeros16 = jnp.zeros((16,), f32)

    for hr in range(2):
        for v in range(8):
            hist_l[hr, pl.ds(v * 16, 16)] = zeros16
    a0 = sid * A_SLICE
    n_full, rem = divmod(A_SLICE, E_WIN)
    for w in range(n_full + (1 if rem else 0)):
        wlen = E_WIN if w < n_full else rem
        a_start = pl.multiple_of(a0 + w * E_WIN, 8)
        pltpu.sync_copy(dst_hbm.at[pl.ds(a_start, wlen)], dstb.at[pl.ds(0, wlen)])

        def abody(i, _):
            d = dstb[pl.ds(i * 16, 16)]
            g = lax.shift_right_logical(d, 3)
            gr = lax.shift_right_logical(g, 7)
            gc = jnp.bitwise_and(g, 127)
            plsc.addupdate_scatter(hist_l, [gr, gc], ones)
            return 0

        lax.fori_loop(0, wlen // 16, abody, 0)
    pltpu.sync_copy(hist_l, hist_sh.at[sid])
    plsc.subcore_barrier()

    @pl.when(sid == 0)
    def _reduce_scan():
        pltpu.sync_copy(hist_sh, hist_all)
        carry = jnp.int32(0)
        for v in range(16):
            hr, hc = v >> 3, (v & 7) * 16
            tot = hist_all[0, hr, pl.ds(hc, 16)]
            for r in range(1, 16):
                tot = tot + hist_all[r, hr, pl.ds(hc, 16)]
            incl = plsc.cumsum(tot)
            excl = (incl - tot).astype(jnp.int32) + carry
            rs_l[pl.ds(v * 16, 16)] = excl
            carry = carry + jnp.sum(tot).astype(jnp.int32)
        rs_l[pl.ds(256, 16)] = jnp.full((16,), 1, jnp.int32) * carry
        pltpu.sync_copy(rs_l, rs_sh)

    plsc.subcore_barrier()
    pltpu.sync_copy(rs_sh, rs_l)

    for j in range(GPT):
        k = wid * GPT + j
        lo = k * G_ROWS

        @pl.when(lo < P)
        def _group():
            s0 = rs_l[pl.ds(k, 16)][0]
            e0 = rs_l[pl.ds(k + 1, 16)][0]
            for r in range(G_ROWS):
                def zbody(cb, _):
                    for u in range(16):
                        cbuf[r, pl.ds((cb * 16 + u) * 16, 16)] = zeros16
                    return 0
                lax.fori_loop(0, N_PAD // 256, zbody, 0)
            base = jnp.bitwise_and(s0, jnp.int32(-8))
            n_win = lax.shift_right_logical(e0 - base + (E_WIN - 1), 10)

            def wbody(w, _):
                start = pl.multiple_of(base + w * E_WIN, 8)
                pltpu.sync_copy(src_hbm.at[pl.ds(start, E_WIN)], srcb)
                pltpu.sync_copy(dst_hbm.at[pl.ds(start, E_WIN)], dstb)

                def ebody(i, _):
                    sv = srcb[pl.ds(i * 16, 16)]
                    dv = dstb[pl.ds(i * 16, 16)]
                    rr = dv - lo
                    m = jnp.logical_and(dv >= lo, dv < lo + G_ROWS)
                    plsc.addupdate_scatter(cbuf, [rr, sv], ones, mask=m)
                    return 0

                lax.fori_loop(0, E_WIN // 16, ebody, 0)
                return 0

            lax.fori_loop(0, n_win, wbody, 0)
            pltpu.sync_copy(cbuf, c_hbm.at[pl.ds(lo, G_ROWS), :])


def _edge_counts(edge_src, edge_dst):
    src_p = jnp.pad(edge_src, (0, E_WIN))
    dst_p = jnp.pad(edge_dst, (0, E_WIN), constant_values=1 << 20)
    return _c_build_sc(src_p, dst_p)


def kernel(node_hidden, energy, node_scalars, isTrack, particle_hidden,
           particle_global, edge_dR, edge_src, edge_dst, W_key, b_key,
           W_val, b_val, W_q, b_q, W_ih, W_hh, b_ih, b_hh, ln_g, ln_b,
           W_m1, b_m1, W_m2, b_m2, rms_w, lin_w):
    f32 = jnp.float32

    scale = jnp.ones((27,), f32)
    shift = jnp.zeros((27,), f32)
    scale = scale.at[0].set(1.0 / TOPO_E_STD).at[1].set(1.0 / ETA_STD)
    shift = shift.at[0].set(TOPO_E_MEAN).at[1].set(ETA_MEAN)
    scale = jax.lax.dynamic_update_slice(
        scale, jnp.full((6,), 1.0 / ETA_STD, f32), (4,))
    shift = jax.lax.dynamic_update_slice(
        shift, jnp.full((6,), ETA_MEAN, f32), (4,))

    def fold(W, b):
        Wskip = W[D:] * scale[:, None]
        b2 = b - (shift * scale) @ W[D:]
        w_e = Wskip[0:1]
        w_t = Wskip[22:23]
        Wsc = jnp.concatenate([Wskip[1:22], Wskip[23:27],
                               jnp.zeros((7, W.shape[1]), f32)], axis=0)
        return W[:D], Wsc, w_e, w_t, b2[None, :]

    def pad_out(x, width=D):
        return jnp.pad(x, ((0, 0), (0, width - x.shape[1])))

    wk1, wk2, wke, wkt, bk = (pad_out(a) for a in fold(W_key, b_key))
    wv1, wv2, wve, wvt, bv = fold(W_val, b_val)

    pad_n = ((0, N_PAD - N), (0, 0))
    nh_p = jnp.pad(node_hidden, pad_n)
    sc_p = jnp.pad(node_scalars, ((0, N_PAD - N), (0, 32 - 25)))
    en_p = jnp.pad(energy[:, None], pad_n, constant_values=1.0)
    tr_p = jnp.pad(isTrack[:, None], pad_n)

    kmat, vmat = _node_projections(nh_p, sc_p, en_p, tr_p,
                                   wk1, wk2, wke, wkt, bk,
                                   wv1, wv2, wve, wvt, bv)

    cmat = _edge_counts(edge_src, edge_dst)

    wq1 = pad_out(W_q[:D])
    wq2 = pad_out(W_q[D:])
    bq = pad_out(b_q[None, :])

    out = _fused_stage(cmat, kmat, vmat, particle_hidden, particle_global,
                       wq1, wq2, bq, W_ih.T, W_hh.T, b_ih[None, :],
                       b_hh[None, :], ln_g[None, :], ln_b[None, :],
                       W_m1, b_m1[None, :], W_m2, b_m2[None, :],
                       rms_w[None, :], lin_w)
    return out
